# Initial kernel scaffold; baseline (speedup 1.0000x reference)
import jax
import jax.numpy as jnp
from jax.experimental import pallas as pl


def kernel(x, edge_index, batch, W1, b1, Wp1, bp1, W2, b2, Wp2, bp2, W3, b3, Wp3, bp3, L1W, L1b, L2W, L2b, L3W, L3b):
    raise NotImplementedError("write your pallas kernel here")


# speedup vs baseline: 34.0731x; 34.0731x over previous
import functools

import jax
import jax.numpy as jnp
from jax import lax
from jax.experimental import pallas as pl
from jax.experimental.pallas import tpu as pltpu
from jax.experimental.pallas import tpu_sc as plsc

N = 10000
NP = 10240
E = 320000
G = 64
D = 128
C = 10
NEG = -1e30

NC = 2
NS = 16
NW = NC * NS
EPT = E // NW
PADL = 10240
K16 = 128
K = 128
NCH = PADL // K
K128 = 80
NCH128 = PADL // K128
RPT = NP // NS

_MESH = plsc.VectorSubcoreMesh(core_axis_name="c", subcore_axis_name="s")
_SC_PARAMS = pltpu.CompilerParams(use_tc_tiling_on_sc=False,
                                  needs_layout_passes=False)

f32 = jnp.float32
i32 = jnp.int32
u32 = jnp.uint32


def _wid():
    return lax.axis_index("c") * NS + lax.axis_index("s"), lax.axis_index("c"), lax.axis_index("s")


def _fill1d(ref, n, val, dtype):
    def body(i, _):
        ref[pl.ds(i * 16, 16)] = jnp.full((16,), val, dtype)
        return 0
    lax.fori_loop(0, n // 16, body, 0)


def _fill2d(ref, rows, width, val, dtype=f32):
    per = width // 16

    def body(i, _):
        r = i // per
        j = i % per
        ref[r, pl.ds(j * 16, 16)] = jnp.full((16,), val, dtype)
        return 0
    lax.fori_loop(0, rows * per, body, 0)


def _lane0(v):
    return jnp.sum(jnp.where(lax.iota(i32, 16) == 0, v, 0))


def _edge_rows_loop(tab_hbm, src_v, dst_v, rowsA, rowsB, acc_sh, semA, semB, nch):
    @pl.when(nch > 0)
    def _():
        pltpu.async_copy(tab_hbm.at[src_v.at[0]], rowsA, semA)

    def pair(p, _):
        i = 2 * p

        @pl.when(i + 1 < nch)
        def _():
            pltpu.async_copy(tab_hbm.at[src_v.at[i + 1]], rowsB, semB)
        pltpu.make_async_copy(tab_hbm.at[src_v.at[i]], rowsA, semA).wait()
        pltpu.sync_copy(rowsA, acc_sh.at[dst_v.at[i]], add=True)

        @pl.when(i + 2 < nch)
        def _():
            pltpu.async_copy(tab_hbm.at[src_v.at[i + 2]], rowsA, semA)

        @pl.when(i + 1 < nch)
        def _():
            pltpu.make_async_copy(tab_hbm.at[src_v.at[i + 1]], rowsB, semB).wait()
            pltpu.sync_copy(rowsB, acc_sh.at[dst_v.at[i + 1]], add=True)
        return 0
    lax.fori_loop(0, (nch + 1) // 2, pair, 0)


@functools.partial(
    pl.kernel, mesh=_MESH, compiler_params=_SC_PARAMS,
    out_type=jax.ShapeDtypeStruct((NW, 16), i32),
    scratch_types=[
        pltpu.VMEM((NP,), i32),
        pltpu.VMEM((16,), i32),
    ])
def _sc_cs(batch_hbm, cs_out, batch_v, csb_v):
    w, _, _ = _wid()
    pltpu.sync_copy(batch_hbm, batch_v)
    g0 = 2 * w
    g1 = g0 + 1

    def scan(i, carry):
        cnt0, cnt1, st0, st1 = carry
        b = batch_v[pl.ds(i * 16, 16)]
        cnt0 = cnt0 + plsc.all_reduce_population_count(b == g0)
        cnt1 = cnt1 + plsc.all_reduce_population_count(b == g1)
        st0 = st0 + plsc.all_reduce_population_count(b < g0)
        st1 = st1 + plsc.all_reduce_population_count(b < g1)
        return (cnt0, cnt1, st0, st1)

    z = jnp.zeros((16,), i32)
    cnt0, cnt1, st0, st1 = lax.fori_loop(0, NP // 16, scan, (z, z, z, z))
    lanes = lax.iota(i32, 16)
    v = jnp.where(lanes == 0, cnt0,
                  jnp.where(lanes == 1, cnt1,
                            jnp.where(lanes == 2, st0,
                                      jnp.where(lanes == 3, st1, 0))))
    csb_v[...] = v
    pltpu.sync_copy(csb_v, cs_out.at[w])


def _make_sc_pass(wd, kk):
    nchw = PADL // kk

    @functools.partial(
        pl.kernel, mesh=_MESH, compiler_params=_SC_PARAMS,
        out_type=jax.ShapeDtypeStruct((NC, NP, wd), f32),
        scratch_types=[
            pltpu.VMEM((nchw, kk), i32),
            pltpu.VMEM((nchw, kk), i32),
            pltpu.VMEM((16,), i32),
            pltpu.VMEM((kk, wd), f32),
            pltpu.VMEM((kk, wd), f32),
            pltpu.VMEM((32, wd), f32),
            pltpu.VMEM_SHARED((NP, wd), f32),
            pltpu.SemaphoreType.DMA,
            pltpu.SemaphoreType.DMA,
        ])
    def _pass(tab_hbm, src3_hbm, dst3_hbm, ecnt_hbm, out_hbm,
              src_v, dst_v, ecb_v, rowsA, rowsB, zb_v, acc_sh, semA, semB):
        w, c, s = _wid()
        pltpu.sync_copy(src3_hbm.at[w], src_v)
        pltpu.sync_copy(dst3_hbm.at[w], dst_v)
        pltpu.sync_copy(ecnt_hbm.at[w], ecb_v)
        nch = (_lane0(ecb_v[...]) + kk - 1) // kk
        _fill2d(zb_v, 32, wd, 0.0)

        def zc(i, _):
            pltpu.sync_copy(zb_v, acc_sh.at[pl.ds(s * RPT + i * 32, 32)])
            return 0
        lax.fori_loop(0, RPT // 32, zc, 0)
        plsc.subcore_barrier()

        _edge_rows_loop(tab_hbm, src_v, dst_v, rowsA, rowsB, acc_sh,
                        semA, semB, nch)

        plsc.subcore_barrier()
        pltpu.sync_copy(acc_sh.at[pl.ds(s * RPT, RPT)],
                        out_hbm.at[c, pl.ds(s * RPT, RPT)])

    return _pass


_sc_pass16 = _make_sc_pass(16, K16)
_sc_pass128 = _make_sc_pass(D, K128)


@functools.partial(
    pl.kernel, mesh=_MESH, compiler_params=_SC_PARAMS,
    out_type=jax.ShapeDtypeStruct((NC, NP, 16), f32),
    scratch_types=[
        pltpu.VMEM((NCH, K), i32),
        pltpu.VMEM((16,), i32),
        pltpu.VMEM((K, 16), f32),
        pltpu.VMEM((64, 16), f32),
        pltpu.VMEM_SHARED((NP, 16), f32),
    ])
def _sc_deg(dst3_hbm, ecnt_hbm, out_hbm, dst_v, ecb_v, ones_v, zb_v, acc_sh):
    w, c, s = _wid()
    pltpu.sync_copy(dst3_hbm.at[w], dst_v)
    pltpu.sync_copy(ecnt_hbm.at[w], ecb_v)
    nch = (_lane0(ecb_v[...]) + K - 1) // K
    lanes = lax.iota(i32, 16)

    def fo(r, _):
        ones_v[r, pl.ds(0, 16)] = jnp.where(lanes == 0, 1.0, 0.0)
        return 0
    lax.fori_loop(0, K, fo, 0)
    _fill2d(zb_v, 64, 16, 0.0)

    def zc(i, _):
        pltpu.sync_copy(zb_v, acc_sh.at[pl.ds(s * RPT + i * 64, 64)])
        return 0
    lax.fori_loop(0, RPT // 64, zc, 0)
    plsc.subcore_barrier()

    def chunk(i, _):
        pltpu.sync_copy(ones_v, acc_sh.at[dst_v.at[i]], add=True)
        return 0
    lax.fori_loop(0, nch, chunk, 0)

    plsc.subcore_barrier()
    pltpu.sync_copy(acc_sh.at[pl.ds(s * RPT, RPT)],
                    out_hbm.at[c, pl.ds(s * RPT, RPT)])


@functools.partial(
    pl.kernel, mesh=_MESH, compiler_params=_SC_PARAMS,
    out_type=jax.ShapeDtypeStruct((NC, NP), f32),
    scratch_types=[
        pltpu.VMEM((NP,), u32),
        pltpu.VMEM((NP,), f32),
        pltpu.VMEM((NP,), f32),
        pltpu.VMEM((16,), i32),
        pltpu.VMEM((16,), i32),
        pltpu.VMEM((16,), f32),
        pltpu.VMEM((RPT,), f32),
        pltpu.VMEM_SHARED((NP,), f32),
    ])
def _sc_topk(ku_hbm, m0_hbm, m1_hbm, cs_hbm, out_hbm,
             ku_v, mt_v, mb_v, csb_v, idxb_v, valb_v, zb_v, sel_sh):
    w, c, s = _wid()
    pltpu.sync_copy(ku_hbm, ku_v)
    pltpu.sync_copy(m0_hbm, mt_v)
    pltpu.sync_copy(m1_hbm, mb_v)
    pltpu.sync_copy(cs_hbm.at[w], csb_v)
    _fill1d(zb_v, RPT, 0.0, f32)

    def comb(i, _):
        mt_v[pl.ds(i * 16, 16)] = mt_v[pl.ds(i * 16, 16)] + mb_v[pl.ds(i * 16, 16)]
        return 0
    lax.fori_loop(0, NP // 16, comb, 0)

    pltpu.sync_copy(zb_v, sel_sh.at[pl.ds(s * RPT, RPT)])
    plsc.subcore_barrier()

    csv = csb_v[...]
    lanes = lax.iota(i32, 16)
    zi = jnp.zeros((16,), i32)

    for gi in range(2):
        cn = jnp.sum(jnp.where(lanes == gi, csv, 0))
        st = jnp.sum(jnp.where(lanes == gi + 2, csv, 0))
        end = st + cn
        c_lo = st // 16
        c_hi = (end + 15) // 16

        def count_pred(cmp):
            def body(ci, acc):
                p0 = ci * 16
                kuc = ku_v[pl.ds(p0, 16)]
                mfc = mt_v[pl.ds(p0, 16)]
                pos = p0 + lanes
                ok = (pos >= st) & (pos < end) & (mfc > 0.5) & cmp(kuc)
                return acc + plsc.all_reduce_population_count(ok)
            return lax.fori_loop(c_lo, c_hi, body, zi)

        cntm = count_pred(lambda kuc: kuc == kuc)
        k_vec = (cntm + 1) >> 1

        def bit_body(bi, t_vec):
            shift = (31 - bi).astype(u32)
            cand = t_vec | lax.shift_left(jnp.uint32(1), shift)
            cge = count_pred(lambda kuc: kuc >= cand)
            return jnp.where(cge >= k_vec, cand, t_vec)
        t_vec = lax.fori_loop(0, 32, bit_body, jnp.zeros((16,), u32))

        cgt = count_pred(lambda kuc: kuc > t_vec)
        r_vec = k_vec - cgt

        def sel_body(ci, runv):
            p0 = ci * 16
            kuc = ku_v[pl.ds(p0, 16)]
            mfc = mt_v[pl.ds(p0, 16)]
            pos = p0 + lanes
            valid = (pos >= st) & (pos < end) & (mfc > 0.5)
            eq = valid & (kuc == t_vec)
            gt = valid & (kuc > t_vec)
            eqi = jnp.where(eq, 1, 0)
            pre = runv + plsc.cumsum(eqi) - eqi
            selv = gt | (eq & (pre < r_vec))
            idxb_v[...] = p0 + lanes
            valb_v[...] = jnp.where(selv, 1.0, 0.0).astype(f32)
            pltpu.sync_copy(valb_v, sel_sh.at[idxb_v], add=True)
            return runv + plsc.all_reduce_population_count(eq)
        lax.fori_loop(c_lo, c_hi, sel_body, zi)

    plsc.subcore_barrier()
    pltpu.sync_copy(sel_sh.at[pl.ds(s * RPT, RPT)],
                    out_hbm.at[c, pl.ds(s * RPT, RPT)])


@functools.partial(
    pl.kernel, mesh=_MESH, compiler_params=_SC_PARAMS,
    out_type=[jax.ShapeDtypeStruct((G, 256), f32),
              jax.ShapeDtypeStruct((NC, NP, 16), f32),
              jax.ShapeDtypeStruct((NW, NCH, K), i32),
              jax.ShapeDtypeStruct((NW, NCH, K), i32),
              jax.ShapeDtypeStruct((NW, 16), i32)],
    scratch_types=[
        pltpu.VMEM((NCH, K), i32),
        pltpu.VMEM((NCH, K), i32),
        pltpu.VMEM((NCH, K), i32),
        pltpu.VMEM((NCH, K), i32),
        pltpu.VMEM((K, 16), f32),
        pltpu.VMEM((NP,), f32),
        pltpu.VMEM((NP,), f32),
        pltpu.VMEM((16, D), f32),
        pltpu.VMEM((16, D), f32),
        pltpu.VMEM((256,), f32),
        pltpu.VMEM((64, 16), f32),
        pltpu.VMEM((16,), i32),
        pltpu.VMEM((16,), i32),
        pltpu.VMEM((16,), i32),
        pltpu.VMEM_SHARED((NP, 16), f32),
    ])
def _sc_readout_deg(hm_hbm, hg_hbm, sel0_hbm, sel1_hbm, cs_hbm,
                    src3_hbm, dst3_hbm, ecnt_hbm,
                    read_out, deg_out, csrc_out, cdst_out, ecnt_out,
                    src_v, dst_v, csrc_v, cdst_v, ones_v, selc_v, selb_v,
                    bufm, bufg, ob_v, zb_v, csb_v, ecb_v, ecob_v, acc_sh):
    w, c, s = _wid()
    pltpu.sync_copy(src3_hbm.at[w], src_v)
    pltpu.sync_copy(dst3_hbm.at[w], dst_v)
    pltpu.sync_copy(sel0_hbm, selc_v)
    pltpu.sync_copy(sel1_hbm, selb_v)
    pltpu.sync_copy(cs_hbm.at[w], csb_v)
    pltpu.sync_copy(ecnt_hbm.at[w], ecb_v)
    lanes = lax.iota(i32, 16)
    _fill2d(zb_v, 64, 16, 0.0)
    _fill2d(csrc_v, NCH, K, NP - 1, i32)
    _fill2d(cdst_v, NCH, K, NP - 1, i32)

    def fo(r, _):
        ones_v[r, pl.ds(0, 16)] = jnp.where(lanes == 0, 1.0, 0.0)
        return 0
    lax.fori_loop(0, K, fo, 0)

    def comb(i, _):
        selc_v[pl.ds(i * 16, 16)] = (selc_v[pl.ds(i * 16, 16)]
                                     + selb_v[pl.ds(i * 16, 16)])
        return 0
    lax.fori_loop(0, NP // 16, comb, 0)

    def zc(i, _):
        pltpu.sync_copy(zb_v, acc_sh.at[pl.ds(s * RPT + i * 64, 64)])
        return 0
    lax.fori_loop(0, RPT // 64, zc, 0)
    plsc.subcore_barrier()

    nch_in = (_lane0(ecb_v[...]) + K - 1) // K

    def comp(idx, offv):
        ch = idx // (K // 16)
        j = idx % (K // 16)
        s16 = src_v[ch, pl.ds(j * 16, 16)]
        d16 = dst_v[ch, pl.ds(j * 16, 16)]
        ss = plsc.load_gather(selc_v, [s16])
        dd = plsc.load_gather(selc_v, [d16])
        keep = (ss > 0.5) & (dd > 0.5)
        ki = jnp.where(keep, 1, 0)
        pos = offv + plsc.cumsum(ki) - ki
        plsc.store_scatter(csrc_v, [pos // K, pos % K], s16, mask=keep)
        plsc.store_scatter(cdst_v, [pos // K, pos % K], d16, mask=keep)
        return offv + plsc.all_reduce_population_count(keep)

    offv = lax.fori_loop(0, nch_in * (K // 16), comp, jnp.zeros((16,), i32))
    cnt_new = _lane0(offv)
    ecob_v[...] = jnp.where(lanes == 0, offv, 0)
    pltpu.sync_copy(ecob_v, ecnt_out.at[w])
    pltpu.sync_copy(csrc_v, csrc_out.at[w])
    pltpu.sync_copy(cdst_v, cdst_out.at[w])

    nch_new = (cnt_new + K - 1) // K

    def dchunk(i, _):
        pltpu.sync_copy(ones_v, acc_sh.at[cdst_v.at[i]], add=True)
        return 0
    lax.fori_loop(0, nch_new, dchunk, 0)

    csv = csb_v[...]
    lanes = lax.iota(i32, 16)
    negv = jnp.full((16,), NEG, f32)
    zf = jnp.zeros((16,), f32)

    for gi in range(2):
        cn = jnp.sum(jnp.where(lanes == gi, csv, 0))
        st = jnp.sum(jnp.where(lanes == gi + 2, csv, 0))
        end = st + cn
        c_lo = st // 16
        c_hi = (end + 15) // 16

        def chunk(ci, carry):
            cntv = carry[0]
            mx = carry[1]
            sm = carry[2]
            p0 = ci * 16
            pltpu.sync_copy(hm_hbm.at[pl.ds(p0, 16)], bufm)
            pltpu.sync_copy(hg_hbm.at[pl.ds(p0, 16)], bufg)
            selc = selc_v[pl.ds(p0, 16)]
            pos = p0 + lanes
            vmask = (pos >= st) & (pos < end) & (selc > 0.5)
            cntv = cntv + plsc.all_reduce_population_count(vmask)
            for r in range(16):
                ok = (p0 + r >= st) & (p0 + r < end)
                mx = tuple(
                    jnp.maximum(mx[j], jnp.where(ok, bufm[r, pl.ds(j * 16, 16)], negv))
                    for j in range(8))
                sm = tuple(
                    sm[j] + jnp.where(ok, bufg[r, pl.ds(j * 16, 16)], zf)
                    for j in range(8))
            return (cntv, mx, sm)

        init = (jnp.zeros((16,), i32),
                tuple(negv for _ in range(8)),
                tuple(zf for _ in range(8)))
        cntv, mx, sm = lax.fori_loop(c_lo, c_hi, chunk, init)

        den = jnp.maximum(cntv.astype(f32), 1.0)
        for j in range(8):
            mxj = jnp.where(mx[j] <= NEG / 2, 0.0, mx[j])
            ob_v[pl.ds(j * 16, 16)] = mxj
            ob_v[pl.ds(128 + j * 16, 16)] = sm[j] / den
        g = 2 * w + gi
        pltpu.sync_copy(ob_v, read_out.at[g])

    plsc.subcore_barrier()
    pltpu.sync_copy(acc_sh.at[pl.ds(s * RPT, RPT)],
                    deg_out.at[c, pl.ds(s * RPT, RPT)])


BT = 1024
_GRID = NP // BT


def _vspec(width):
    return pl.BlockSpec((BT, width), lambda i: (i, 0))


def _wspec(r, cc):
    return pl.BlockSpec((r, cc), lambda i: (0, 0))


def _tc_prep_body(x_ref, w_ref, i0_ref, i1_ref, mf0_ref, mf1_ref,
                  xw_ref, ym_ref, deg_ref, dinv_ref):
    mf = mf0_ref[...] + mf1_ref[...]
    deg = 1.0 + mf * (i0_ref[...] + i1_ref[...])
    dinv = lax.rsqrt(deg)
    xw = jnp.dot(x_ref[...], w_ref[...], preferred_element_type=f32)
    xw_ref[...] = xw
    ym_ref[...] = dinv * mf * xw
    deg_ref[...] = deg
    dinv_ref[...] = dinv


def _tc_prep(x, W, i0, i1, mf0, mf1):
    return pl.pallas_call(
        _tc_prep_body,
        grid=(_GRID,),
        in_specs=[_vspec(D), _wspec(D, D), _vspec(1), _vspec(1), _vspec(1), _vspec(1)],
        out_specs=[_vspec(D), _vspec(D), _vspec(1), _vspec(1)],
        out_shape=[jax.ShapeDtypeStruct((NP, D), f32),
                   jax.ShapeDtypeStruct((NP, D), f32),
                   jax.ShapeDtypeStruct((NP, 1), f32),
                   jax.ShapeDtypeStruct((NP, 1), f32)],
    )(x, W, i0, i1, mf0, mf1)


def _tc_conv_body(xw_ref, m0_ref, m1_ref, deg_ref, dinv_ref, b_ref, wp_ref,
                  bp_ref, mf0_ref, mf1_ref, h_ref, uw_ref, xd_ref):
    deg = deg_ref[...]
    dinv = dinv_ref[...]
    conv = dinv * (m0_ref[...] + m1_ref[...]) + xw_ref[...] / deg + b_ref[...]
    h = jnp.maximum(conv, 0.0)
    h_ref[...] = h
    xwp = jnp.dot(h, wp_ref[...], preferred_element_type=f32)
    mf = mf0_ref[...] + mf1_ref[...]
    u = dinv * mf * xwp
    col = lax.broadcasted_iota(i32, (BT, 16), 1)
    uw_ref[...] = jnp.where(col == 0, u, 0.0)
    xd_ref[...] = xwp / deg + bp_ref[...]


def _tc_conv(xw, m0, m1, deg, dinv, b, Wp, bp, mf0, mf1):
    return pl.pallas_call(
        _tc_conv_body,
        grid=(_GRID,),
        in_specs=[_vspec(D), _vspec(D), _vspec(D), _vspec(1), _vspec(1),
                  _wspec(1, D), _wspec(D, 1), _wspec(1, 1), _vspec(1), _vspec(1)],
        out_specs=[_vspec(D), _vspec(16), _vspec(1)],
        out_shape=[jax.ShapeDtypeStruct((NP, D), f32),
                   jax.ShapeDtypeStruct((NP, 16), f32),
                   jax.ShapeDtypeStruct((NP, 1), f32)],
    )(xw, m0, m1, deg, dinv, b, Wp, bp, mf0, mf1)


def _tc_score_body(s0_ref, s1_ref, dinv_ref, xd_ref, mf0_ref, mf1_ref,
                   score_ref, ku_ref):
    p = dinv_ref[...] * (s0_ref[...] + s1_ref[...]) + xd_ref[...]
    score_ref[...] = jnp.tanh(p)
    mf = mf0_ref[...] + mf1_ref[...]
    key = jnp.where(mf > 0.5, p, NEG)
    bu = lax.bitcast_convert_type(key, u32)
    msk = jnp.where(bu >= jnp.uint32(0x80000000),
                    jnp.uint32(0xFFFFFFFF), jnp.uint32(0x80000000))
    ku_ref[...] = bu ^ msk


def _tc_score(s0, s1, dinv, xd, mf0, mf1):
    return pl.pallas_call(
        _tc_score_body,
        grid=(_GRID,),
        in_specs=[_vspec(1)] * 6,
        out_specs=[_vspec(1), _vspec(1)],
        out_shape=[jax.ShapeDtypeStruct((NP, 1), f32),
                   jax.ShapeDtypeStruct((NP, 1), u32)],
    )(s0, s1, dinv, xd, mf0, mf1)


def _tc_gate_body(h_ref, score_ref, sel0_ref, sel1_ref, hg_ref, hm_ref):
    sel = sel0_ref[...] + sel1_ref[...]
    gat = jnp.where(sel > 0.5, score_ref[...], 0.0)
    hg = h_ref[...] * gat
    hg_ref[...] = hg
    hm_ref[...] = jnp.where(sel > 0.5, hg, NEG)


def _tc_gate(h, score, sel0, sel1):
    return pl.pallas_call(
        _tc_gate_body,
        grid=(_GRID,),
        in_specs=[_vspec(D), _vspec(1), _vspec(1), _vspec(1)],
        out_specs=[_vspec(D), _vspec(D)],
        out_shape=[jax.ShapeDtypeStruct((NP, D), f32),
                   jax.ShapeDtypeStruct((NP, D), f32)],
    )(h, score, sel0, sel1)


def _tc_final_body(x1_ref, x2_ref, x3_ref, w1_ref, b1_ref, w2_ref, b2_ref,
                   w3_ref, b3_ref, out_ref):
    sacc = x1_ref[...] + x2_ref[...] + x3_ref[...]
    a = jnp.maximum(jnp.dot(sacc, w1_ref[...], preferred_element_type=f32)
                    + b1_ref[...], 0.0)
    b = jnp.maximum(jnp.dot(a, w2_ref[...], preferred_element_type=f32)
                    + b2_ref[...], 0.0)
    z = jnp.dot(b, w3_ref[...], preferred_element_type=f32) + b3_ref[...]
    m = jnp.max(z, axis=-1, keepdims=True)
    lse = jnp.log(jnp.sum(jnp.exp(z - m), axis=-1, keepdims=True))
    out_ref[...] = z - m - lse


def _tc_final(x1, x2, x3, L1W, L1b, L2W, L2b, L3W, L3b):
    return pl.pallas_call(
        _tc_final_body,
        out_shape=jax.ShapeDtypeStruct((G, C), f32),
    )(x1, x2, x3, L1W, L1b, L2W, L2b, L3W, L3b)


def kernel(x, edge_index, batch, W1, b1, Wp1, bp1, W2, b2, Wp2, bp2,
           W3, b3, Wp3, bp3, L1W, L1b, L2W, L2b, L3W, L3b):
    xp = jnp.zeros((NP, D), f32).at[:N].set(x)
    batch_p = jnp.full((NP,), G, i32).at[:N].set(batch)
    epad = jnp.full((NW, PADL - EPT), NP - 1, i32)
    src3 = jnp.concatenate(
        [edge_index[0].reshape(NW, EPT), epad], axis=1).reshape(NW, NCH, K)
    dst3 = jnp.concatenate(
        [edge_index[1].reshape(NW, EPT), epad], axis=1).reshape(NW, NCH, K)

    ones1 = jnp.ones((NP, 1), f32)
    zeros1 = jnp.zeros((NP, 1), f32)
    onesf = jnp.ones((NP,), f32)
    zerosf = jnp.zeros((NP,), f32)
    ecnt_full = jnp.full((NW, 16), EPT, i32)

    cs = _sc_cs(batch_p)
    indeg = _sc_deg(dst3, ecnt_full)

    layers = [(W1, b1, Wp1, bp1), (W2, b2, Wp2, bp2), (W3, b3, Wp3, bp3)]
    hcur = xp
    mf0, mf1 = ones1, zeros1
    m0f, m1f = onesf, zerosf
    srcc, dstc, ecnt = src3, dst3, ecnt_full
    srcc128 = srcc.reshape(NW, NCH128, K128)
    dstc128 = dstc.reshape(NW, NCH128, K128)
    i0 = indeg[0, :, 0].reshape(NP, 1)
    i1 = indeg[1, :, 0].reshape(NP, 1)
    reads = []

    for li in range(3):
        W, b, Wp, bp = layers[li]
        xw, ym, deg, dinv = _tc_prep(hcur, W, i0, i1, mf0, mf1)
        msg = _sc_pass128(ym, srcc128, dstc128, ecnt)
        h, uw, xd = _tc_conv(xw, msg[0], msg[1], deg, dinv,
                             b.reshape(1, D), Wp, bp.reshape(1, 1), mf0, mf1)
        smsg = _sc_pass16(uw, srcc, dstc, ecnt)
        score, ku = _tc_score(smsg[0, :, 0].reshape(NP, 1),
                              smsg[1, :, 0].reshape(NP, 1), dinv, xd, mf0, mf1)
        selp = _sc_topk(ku.reshape(NP), m0f, m1f, cs)
        hg, hm = _tc_gate(h, score,
                          selp[0].reshape(NP, 1), selp[1].reshape(NP, 1))
        xr, indeg_n, srcc, dstc, ecnt = _sc_readout_deg(
            hm, hg, selp[0], selp[1], cs, srcc, dstc, ecnt)
        srcc128 = srcc.reshape(NW, NCH128, K128)
        dstc128 = dstc.reshape(NW, NCH128, K128)
        reads.append(xr)
        hcur = hg
        mf0, mf1 = selp[0].reshape(NP, 1), selp[1].reshape(NP, 1)
        m0f, m1f = selp[0], selp[1]
        i0 = indeg_n[0, :, 0].reshape(NP, 1)
        i1 = indeg_n[1, :, 0].reshape(NP, 1)

    return _tc_final(reads[0], reads[1], reads[2],
                     L1W, L1b.reshape(1, -1), L2W, L2b.reshape(1, -1),
                     L3W, L3b.reshape(1, -1))
